# baseline (speedup 1.0000x reference)
import functools

import jax
import jax.numpy as jnp
from jax import lax
from jax.experimental import pallas as pl
from jax.experimental.pallas import tpu as pltpu
from jax.experimental.pallas import tpu_sc as plsc

_B = 1024
_S = 50
_D = 1000
_NW = 32
_RPW = _B // _NW
_GRP = _RPW // 16

_mesh = plsc.VectorSubcoreMesh(core_axis_name="c", subcore_axis_name="s")


@functools.partial(
    pl.kernel,
    mesh=_mesh,
    out_type=jax.ShapeDtypeStruct((_B, _D), jnp.float32),
    compiler_params=pltpu.CompilerParams(needs_layout_passes=False),
    scratch_types=[
        pltpu.VMEM((_RPW, _S), jnp.float32),
        pltpu.VMEM((_RPW, _D), jnp.float32),
    ],
)
def _hist_kernel(in_hbm, out_hbm, in_v, out_v):
    wid = lax.axis_index("s") * 2 + lax.axis_index("c")
    base = wid * _RPW

    pltpu.sync_copy(in_hbm.at[pl.ds(base, _RPW)], in_v)

    zeros = jnp.zeros((16,), jnp.float32)
    _COLS = tuple(range(0, _D - 16, 16)) + (_D - 16,)

    def zbody(r, carry):
        for c in _COLS:
            out_v[r, pl.ds(c, 16)] = zeros
        return carry

    lax.fori_loop(0, _RPW, zbody, 0, unroll=False)

    lanes = lax.iota(jnp.int32, 16)
    ones = jnp.ones((16,), jnp.float32)

    for g in range(_GRP):
        rows = lanes + g * 16
        for c in range(_S):
            col = jnp.full((16,), c, jnp.int32)
            vals = plsc.load_gather(in_v, [rows, col])
            plsc.addupdate_scatter(out_v, [rows, vals.astype(jnp.int32)], ones)

    pltpu.sync_copy(out_v, out_hbm.at[pl.ds(base, _RPW)])


def kernel(inputs):
    return _hist_kernel(inputs)


# speedup vs baseline: 1.1895x; 1.1895x over previous
import functools

import jax
import jax.numpy as jnp
from jax import lax
from jax.experimental import pallas as pl
from jax.experimental.pallas import tpu as pltpu
from jax.experimental.pallas import tpu_sc as plsc

_B = 1024
_S = 50
_D = 1000
_STRIPE = 128
_Q = 256
_QLAST = _D - 3 * _Q

_mesh = plsc.VectorSubcoreMesh(core_axis_name="c", subcore_axis_name="s")


@functools.partial(
    pl.kernel,
    mesh=_mesh,
    out_type=jax.ShapeDtypeStruct((_D, _B), jnp.float32),
    compiler_params=pltpu.CompilerParams(needs_layout_passes=False),
    scratch_types=[
        pltpu.VMEM((_S, _STRIPE), jnp.float32),
        pltpu.VMEM((_Q, _STRIPE), jnp.float32),
    ],
)
def _hist_kernel(in_hbm, out_hbm, in_v, out_v):
    wid = lax.axis_index("s") * 2 + lax.axis_index("c")
    stripe_base = (wid // 4) * _STRIPE
    q = wid % 4
    depth_base = q * _Q

    pltpu.sync_copy(in_hbm.at[:, pl.ds(stripe_base, _STRIPE)], in_v)

    zeros = jnp.zeros((16,), jnp.float32)

    def zbody(i, carry):
        r = pl.multiple_of(i * 4, 4)
        for k in range(4):
            for c in range(0, _STRIPE, 16):
                out_v[r + k, pl.ds(c, 16)] = zeros
        return carry

    lax.fori_loop(0, _Q // 4, zbody, 0, unroll=False)

    lanes = lax.iota(jnp.int32, 16)
    ones = jnp.ones((16,), jnp.float32)

    def cbody(c, carry):
        for g in range(_STRIPE // 16):
            rows = lanes + g * 16
            vals = in_v[c, pl.ds(g * 16, 16)].astype(jnp.int32) - depth_base
            mask = (vals >= 0) & (vals < _Q)
            plsc.addupdate_scatter(out_v, [vals, rows], ones, mask=mask)
        return carry

    lax.fori_loop(0, _S, cbody, 0, unroll=False)

    @pl.when(q < 3)
    def _():
        pltpu.sync_copy(
            out_v, out_hbm.at[pl.ds(depth_base, _Q), pl.ds(stripe_base, _STRIPE)]
        )

    @pl.when(q == 3)
    def _():
        pltpu.sync_copy(
            out_v.at[pl.ds(0, _QLAST)],
            out_hbm.at[pl.ds(3 * _Q, _QLAST), pl.ds(stripe_base, _STRIPE)],
        )


def kernel(inputs):
    return _hist_kernel(inputs.T).T


# speedup vs baseline: 1.3307x; 1.1187x over previous
import functools

import jax
import jax.numpy as jnp
from jax import lax
from jax.experimental import pallas as pl
from jax.experimental.pallas import tpu as pltpu
from jax.experimental.pallas import tpu_sc as plsc

_B = 1024
_S = 50
_D = 1000
_STRIPE = 128
_Q = 256
_QLAST = _D - 3 * _Q

_mesh = plsc.VectorSubcoreMesh(core_axis_name="c", subcore_axis_name="s")


@functools.partial(
    pl.kernel,
    mesh=_mesh,
    out_type=jax.ShapeDtypeStruct((_D, _B), jnp.float32),
    compiler_params=pltpu.CompilerParams(needs_layout_passes=False),
    scratch_types=[
        pltpu.VMEM((_S, _STRIPE), jnp.float32),
        pltpu.VMEM((_Q, _STRIPE), jnp.float32),
    ],
)
def _hist_kernel(in_hbm, out_hbm, in_v, out_v):
    wid = lax.axis_index("s") * 2 + lax.axis_index("c")
    stripe_base = (wid // 4) * _STRIPE
    q = wid % 4
    depth_base = q * _Q

    pltpu.sync_copy(in_hbm.at[:, pl.ds(stripe_base, _STRIPE)], in_v)

    zeros = jnp.zeros((16,), jnp.float32)

    @plsc.parallel_loop(0, _Q, step=4)
    def _(i):
        r = pl.multiple_of(i, 4)
        for k in range(4):
            for c in range(0, _STRIPE, 16):
                out_v[r + k, pl.ds(c, 16)] = zeros

    lanes = lax.iota(jnp.int32, 16)
    ones = jnp.ones((16,), jnp.float32)

    @plsc.parallel_loop(0, _S, step=1)
    def _(c):
        for g in range(_STRIPE // 16):
            rows = lanes + g * 16
            vals = in_v[c, pl.ds(g * 16, 16)].astype(jnp.int32)
            mask = lax.shift_right_logical(vals, 8) == q
            local = lax.bitwise_and(vals, _Q - 1)
            plsc.addupdate_scatter(out_v, [local, rows], ones, mask=mask)

    @pl.when(q < 3)
    def _():
        pltpu.sync_copy(
            out_v, out_hbm.at[pl.ds(depth_base, _Q), pl.ds(stripe_base, _STRIPE)]
        )

    @pl.when(q == 3)
    def _():
        pltpu.sync_copy(
            out_v.at[pl.ds(0, _QLAST)],
            out_hbm.at[pl.ds(3 * _Q, _QLAST), pl.ds(stripe_base, _STRIPE)],
        )


def kernel(inputs):
    return _hist_kernel(inputs.T).T
